# Initial kernel scaffold; baseline (speedup 1.0000x reference)
import jax
import jax.numpy as jnp
from jax.experimental import pallas as pl


def kernel(x, weight):
    raise NotImplementedError("write your pallas kernel here")


# speedup vs baseline: 5.5570x; 5.5570x over previous
import jax
import jax.numpy as jnp
from jax.experimental import pallas as pl
from jax.experimental.pallas import tpu as pltpu

DENSE_L = 20
STD = 0.4
L_TILDE = 16
P_SUPER = 32
KBLK = P_SUPER * DENSE_L
NBLK = P_SUPER * L_TILDE


def _build_w_kernel(wcol_ref, w_ref):
    r = jax.lax.broadcasted_iota(jnp.int32, (KBLK, NBLK), 0)
    c = jax.lax.broadcasted_iota(jnp.int32, (KBLK, NBLK), 1)
    pr = r // DENSE_L
    pc = c // L_TILDE
    t = (r % DENSE_L + 1).astype(jnp.float32)
    wc = jnp.broadcast_to(wcol_ref[...], (KBLK, NBLK))
    g = jnp.exp(-((t - wc) ** 2) * (1.0 / (STD * STD)))
    w_ref[...] = jnp.where(pr == pc, g, 0.0)


def _contract_kernel(x_ref, w_ref, o_ref):
    w = w_ref[...]
    n_super = x_ref.shape[1] // KBLK
    for s in range(n_super):
        o_ref[:, s * NBLK:(s + 1) * NBLK] = jnp.dot(
            x_ref[:, s * KBLK:(s + 1) * KBLK], w,
            preferred_element_type=jnp.float32)


def kernel(x, weight):
    B, C = x.shape
    P = C // DENSE_L
    OUTC = P * L_TILDE

    wcol = jnp.tile(weight, P_SUPER).reshape(1, NBLK)

    wblk = pl.pallas_call(
        _build_w_kernel,
        out_shape=jax.ShapeDtypeStruct((KBLK, NBLK), jnp.float32),
        name="gauss_basis_build",
    )(wcol)

    BM = 512
    out = pl.pallas_call(
        _contract_kernel,
        grid=(B // BM,),
        in_specs=[
            pl.BlockSpec((BM, C), lambda i: (i, 0)),
            pl.BlockSpec((KBLK, NBLK), lambda i: (0, 0)),
        ],
        out_specs=pl.BlockSpec((BM, OUTC), lambda i: (i, 0)),
        out_shape=jax.ShapeDtypeStruct((B, OUTC), jnp.float32),
        compiler_params=pltpu.CompilerParams(
            dimension_semantics=("parallel",),
            vmem_limit_bytes=56 * 1024 * 1024,
        ),
        name="gauss_segment_contract",
    )(x, wblk)
    return out


# speedup vs baseline: 5.7719x; 1.0387x over previous
import jax
import jax.numpy as jnp
from jax.experimental import pallas as pl
from jax.experimental.pallas import tpu as pltpu

DENSE_L = 20
STD = 0.4
L_TILDE = 16
P_SUPER = 32
KBLK = P_SUPER * DENSE_L
NBLK = P_SUPER * L_TILDE


def _build_wblk(wcol):
    r = jax.lax.broadcasted_iota(jnp.int32, (KBLK, NBLK), 0)
    c = jax.lax.broadcasted_iota(jnp.int32, (KBLK, NBLK), 1)
    pr = r // DENSE_L
    pc = c // L_TILDE
    t = (r % DENSE_L + 1).astype(jnp.float32)
    wc = jnp.broadcast_to(wcol, (KBLK, NBLK))
    g = jnp.exp(-((t - wc) ** 2) * (1.0 / (STD * STD)))
    return jnp.where(pr == pc, g, 0.0)


def _contract_kernel(x_ref, wcol_ref, o_ref):
    w = _build_wblk(wcol_ref[...])
    n_super = x_ref.shape[1] // KBLK
    for s in range(n_super):
        o_ref[:, s * NBLK:(s + 1) * NBLK] = jnp.dot(
            x_ref[:, s * KBLK:(s + 1) * KBLK], w,
            preferred_element_type=jnp.float32)


def kernel(x, weight):
    B, C = x.shape
    P = C // DENSE_L
    OUTC = P * L_TILDE

    wcol = jnp.tile(weight, P_SUPER).reshape(1, NBLK)

    BM = 512
    out = pl.pallas_call(
        _contract_kernel,
        grid=(B // BM,),
        in_specs=[
            pl.BlockSpec((BM, C), lambda i: (i, 0)),
            pl.BlockSpec((1, NBLK), lambda i: (0, 0)),
        ],
        out_specs=pl.BlockSpec((BM, OUTC), lambda i: (i, 0)),
        out_shape=jax.ShapeDtypeStruct((B, OUTC), jnp.float32),
        compiler_params=pltpu.CompilerParams(
            dimension_semantics=("parallel",),
            vmem_limit_bytes=56 * 1024 * 1024,
        ),
        name="gauss_segment_contract",
    )(x, wcol)
    return out
